# baseline (speedup 1.0000x reference)
import functools

import jax
import jax.numpy as jnp
from jax import lax
from jax.experimental import pallas as pl
from jax.experimental.pallas import tpu as pltpu
from jax.experimental.pallas import tpu_sc as plsc

MAP_H, MAP_W = 32, 32
INPUT_DIM = 256
BATCH = 256
LR = 0.1
RADIUS = 2.0
NCELLS = MAP_H * MAP_W

SC_CORES = 2
SC_SUBCORES = 16
LANES = 16
NWORKERS = SC_CORES * SC_SUBCORES
ROWS_PER_W = BATCH // NWORKERS


def _scores_body(wf_ref, x_ref, scores_ref):
    wf = wf_ref[...]
    x = x_ref[...]
    xw = lax.dot_general(x, wf, (((1,), (1,)), ((), ())),
                         preferred_element_type=jnp.float32,
                         precision=lax.Precision.HIGHEST)
    wn_row = lax.dot_general(jnp.ones((1, INPUT_DIM), jnp.float32), wf * wf,
                             (((1,), (1,)), ((), ())),
                             preferred_element_type=jnp.float32,
                             precision=lax.Precision.HIGHEST)
    scores_ref[...] = wn_row - 2.0 * xw


def _lane_shuffle(v, perm):
    return lax.gather(
        v, perm[:, None],
        lax.GatherDimensionNumbers(offset_dims=(), collapsed_slice_dims=(0,),
                                   start_index_map=(0,)),
        (1,), mode=lax.GatherScatterMode.PROMISE_IN_BOUNDS)


def _sc_bmu_gather_body(scores_hbm, aflat_hbm, p_hbm, q_hbm,
                        blk_v, a_v, p_v, q_v, sem_a, sem_b):
    wid = lax.axis_index("s") * SC_CORES + lax.axis_index("c")
    lane = lax.iota(jnp.int32, 16)

    with jax.named_scope("sc_dma_in"):
        cp_a = pltpu.make_async_copy(aflat_hbm, a_v, sem_a)
        cp_b = pltpu.make_async_copy(scores_hbm.at[wid], blk_v, sem_b)
        cp_a.start()
        cp_b.start()
        cp_a.wait()
        cp_b.wait()

    for s in range(ROWS_PER_W):
        mv = [jnp.full((16,), jnp.inf, jnp.float32) for _ in range(4)]
        mc = [jnp.zeros((16,), jnp.int32) for _ in range(4)]
        for c in range(NCELLS // 16):
            a = c % 4
            v = blk_v[s, c // 8, pl.ds((c % 8) * 16, 16)]
            upd = v < mv[a]
            mv[a] = jnp.where(upd, v, mv[a])
            mc[a] = jnp.where(upd, c, mc[a])
        def merge(p0, p1):
            v0, c0 = p0
            v1, c1 = p1
            take = (v1 < v0) | ((v1 == v0) & (c1 < c0))
            return jnp.where(take, v1, v0), jnp.where(take, c1, c0)

        mvr, mcr = merge(merge((mv[0], mc[0]), (mv[1], mc[1])),
                         merge((mv[2], mc[2]), (mv[3], mc[3])))
        mj = mcr * 16 + lane
        for k in (1, 2, 4, 8):
            perm = jnp.bitwise_xor(lane, k)
            ov = _lane_shuffle(mvr, perm)
            oj = _lane_shuffle(mj, perm)
            take = (ov < mvr) | ((ov == mvr) & (oj < mj))
            mvr = jnp.where(take, ov, mvr)
            mj = jnp.where(take, oj, mj)
        by = mj // MAP_W
        bx = mj % MAP_W
        p_v[s, pl.ds(0, 16)] = plsc.load_gather(a_v, [by * MAP_W + lane])
        p_v[s, pl.ds(16, 16)] = plsc.load_gather(a_v, [by * MAP_W + 16 + lane])
        q_v[s, pl.ds(0, 16)] = plsc.load_gather(a_v, [bx * MAP_W + lane])
        q_v[s, pl.ds(16, 16)] = plsc.load_gather(a_v, [bx * MAP_W + 16 + lane])

    with jax.named_scope("sc_dma_out"):
        pltpu.sync_copy(p_v, p_hbm.at[pl.ds(wid * ROWS_PER_W, ROWS_PER_W)])
        pltpu.sync_copy(q_v, q_hbm.at[pl.ds(wid * ROWS_PER_W, ROWS_PER_W)])


def _tail_body(p_ref, q_ref, thr_ref, out_ref):
    mp = 0.5 * lax.dot_general(p_ref[...], q_ref[...],
                               (((0,), (0,)), ((), ())),
                               preferred_element_type=jnp.float32,
                               precision=lax.Precision.HIGHEST)
    out_ref[...] = (mp > thr_ref[...]).astype(jnp.float32)


@jax.jit
def kernel(spike_input, weights, spike_threshold):
    wf = weights.reshape(NCELLS, INPUT_DIM)
    u = jnp.arange(MAP_H, dtype=jnp.float32)
    d = u[:, None] - u[None, :]
    a_flat = jnp.exp(-(d * d) / (2.0 * RADIUS * RADIUS)).reshape(NCELLS)

    scores = pl.pallas_call(
        _scores_body,
        out_shape=jax.ShapeDtypeStruct((BATCH, NCELLS), jnp.float32),
    )(wf, spike_input)
    scores_blk = scores.reshape(NWORKERS, ROWS_PER_W, NCELLS // 128, 128)

    mesh = plsc.VectorSubcoreMesh(
        core_axis_name="c", subcore_axis_name="s",
        num_cores=SC_CORES, num_subcores=SC_SUBCORES)
    p_rows, q_rows = pl.kernel(
        _sc_bmu_gather_body,
        out_type=(jax.ShapeDtypeStruct((BATCH, MAP_H), jnp.float32),
                  jax.ShapeDtypeStruct((BATCH, MAP_W), jnp.float32)),
        mesh=mesh,
        compiler_params=pltpu.CompilerParams(needs_layout_passes=False),
        scratch_types=[
            pltpu.VMEM((ROWS_PER_W, NCELLS // 128, 128), jnp.float32),
            pltpu.VMEM((NCELLS,), jnp.float32),
            pltpu.VMEM((ROWS_PER_W, MAP_H), jnp.float32),
            pltpu.VMEM((ROWS_PER_W, MAP_W), jnp.float32),
            pltpu.SemaphoreType.DMA,
            pltpu.SemaphoreType.DMA,
        ],
    )(scores_blk, a_flat)

    return pl.pallas_call(
        _tail_body,
        out_shape=jax.ShapeDtypeStruct((MAP_H, MAP_W), jnp.float32),
    )(p_rows, q_rows, spike_threshold)


# speedup vs baseline: 1.0995x; 1.0995x over previous
import functools

import jax
import jax.numpy as jnp
from jax import lax
from jax.experimental import pallas as pl
from jax.experimental.pallas import tpu as pltpu
from jax.experimental.pallas import tpu_sc as plsc

MAP_H, MAP_W = 32, 32
INPUT_DIM = 256
BATCH = 256
LR = 0.1
RADIUS = 2.0
NCELLS = MAP_H * MAP_W

SC_CORES = 2
SC_SUBCORES = 16
LANES = 16
NWORKERS = SC_CORES * SC_SUBCORES
ROWS_PER_W = BATCH // NWORKERS


def _scores_body(wf_ref, x_ref, scores_ref):
    wf = wf_ref[...]
    x = x_ref[...]
    xw = lax.dot_general(x, wf, (((1,), (1,)), ((), ())),
                         preferred_element_type=jnp.float32,
                         precision=lax.Precision.HIGHEST)
    wn_row = lax.dot_general(jnp.ones((1, INPUT_DIM), jnp.float32), wf * wf,
                             (((1,), (1,)), ((), ())),
                             preferred_element_type=jnp.float32,
                             precision=lax.Precision.HIGHEST)
    scores_ref[...] = wn_row - 2.0 * xw


def _lane_shuffle(v, perm):
    return lax.gather(
        v, perm[:, None],
        lax.GatherDimensionNumbers(offset_dims=(), collapsed_slice_dims=(0,),
                                   start_index_map=(0,)),
        (1,), mode=lax.GatherScatterMode.PROMISE_IN_BOUNDS)


def _sc_bmu_gather_body(scores_hbm, aflat_hbm, p_hbm, q_hbm,
                        blk_v, a_v, p_v, q_v, spmem_s, spmem_a, sem_a, sem_b):
    cid = lax.axis_index("c")
    sid = lax.axis_index("s")
    wid = cid * SC_SUBCORES + sid
    lane = lax.iota(jnp.int32, 16)

    with jax.named_scope("sc_dma_in"):
        @pl.when(sid == 0)
        def _():
            cp_s = pltpu.make_async_copy(
                scores_hbm.at[pl.ds(cid * SC_SUBCORES, SC_SUBCORES)],
                spmem_s, sem_a)
            cp_a = pltpu.make_async_copy(aflat_hbm, spmem_a, sem_b)
            cp_s.start()
            cp_a.start()
            cp_s.wait()
            cp_a.wait()
        plsc.subcore_barrier()
        cp_b = pltpu.make_async_copy(spmem_s.at[sid], blk_v, sem_a)
        cp_v = pltpu.make_async_copy(spmem_a, a_v, sem_b)
        cp_b.start()
        cp_v.start()
        cp_b.wait()
        cp_v.wait()

    for s in range(ROWS_PER_W):
        mv = [jnp.full((16,), jnp.inf, jnp.float32) for _ in range(4)]
        mc = [jnp.zeros((16,), jnp.int32) for _ in range(4)]
        for c in range(NCELLS // 16):
            a = c % 4
            v = blk_v[s, pl.ds(c * 16, 16)]
            upd = v < mv[a]
            mv[a] = jnp.where(upd, v, mv[a])
            mc[a] = jnp.where(upd, c, mc[a])
        def merge(p0, p1):
            v0, c0 = p0
            v1, c1 = p1
            take = (v1 < v0) | ((v1 == v0) & (c1 < c0))
            return jnp.where(take, v1, v0), jnp.where(take, c1, c0)

        mvr, mcr = merge(merge((mv[0], mc[0]), (mv[1], mc[1])),
                         merge((mv[2], mc[2]), (mv[3], mc[3])))
        mj = mcr * 16 + lane
        for k in (1, 2, 4, 8):
            perm = jnp.bitwise_xor(lane, k)
            ov = _lane_shuffle(mvr, perm)
            oj = _lane_shuffle(mj, perm)
            take = (ov < mvr) | ((ov == mvr) & (oj < mj))
            mvr = jnp.where(take, ov, mvr)
            mj = jnp.where(take, oj, mj)
        by = mj // MAP_W
        bx = mj % MAP_W
        p_v[s, pl.ds(0, 16)] = plsc.load_gather(a_v, [by * MAP_W + lane])
        p_v[s, pl.ds(16, 16)] = plsc.load_gather(a_v, [by * MAP_W + 16 + lane])
        q_v[s, pl.ds(0, 16)] = plsc.load_gather(a_v, [bx * MAP_W + lane])
        q_v[s, pl.ds(16, 16)] = plsc.load_gather(a_v, [bx * MAP_W + 16 + lane])

    with jax.named_scope("sc_dma_out"):
        pltpu.sync_copy(p_v, p_hbm.at[pl.ds(wid * ROWS_PER_W, ROWS_PER_W)])
        pltpu.sync_copy(q_v, q_hbm.at[pl.ds(wid * ROWS_PER_W, ROWS_PER_W)])


def _tail_body(p_ref, q_ref, thr_ref, out_ref):
    mp = 0.5 * lax.dot_general(p_ref[...], q_ref[...],
                               (((0,), (0,)), ((), ())),
                               preferred_element_type=jnp.float32,
                               precision=lax.Precision.HIGHEST)
    out_ref[...] = (mp > thr_ref[...]).astype(jnp.float32)


@jax.jit
def kernel(spike_input, weights, spike_threshold):
    wf = weights.reshape(NCELLS, INPUT_DIM)
    u = jnp.arange(MAP_H, dtype=jnp.float32)
    d = u[:, None] - u[None, :]
    a_flat = jnp.exp(-(d * d) / (2.0 * RADIUS * RADIUS)).reshape(NCELLS)

    scores = pl.pallas_call(
        _scores_body,
        out_shape=jax.ShapeDtypeStruct((BATCH, NCELLS), jnp.float32),
    )(wf, spike_input)
    scores_blk = scores.reshape(NWORKERS, ROWS_PER_W, NCELLS)

    mesh = plsc.VectorSubcoreMesh(
        core_axis_name="c", subcore_axis_name="s",
        num_cores=SC_CORES, num_subcores=SC_SUBCORES)
    p_rows, q_rows = pl.kernel(
        _sc_bmu_gather_body,
        out_type=(jax.ShapeDtypeStruct((BATCH, MAP_H), jnp.float32),
                  jax.ShapeDtypeStruct((BATCH, MAP_W), jnp.float32)),
        mesh=mesh,
        compiler_params=pltpu.CompilerParams(needs_layout_passes=False),
        scratch_types=[
            pltpu.VMEM((ROWS_PER_W, NCELLS), jnp.float32),
            pltpu.VMEM((NCELLS,), jnp.float32),
            pltpu.VMEM((ROWS_PER_W, MAP_H), jnp.float32),
            pltpu.VMEM((ROWS_PER_W, MAP_W), jnp.float32),
            pltpu.VMEM_SHARED((SC_SUBCORES, ROWS_PER_W, NCELLS), jnp.float32),
            pltpu.VMEM_SHARED((NCELLS,), jnp.float32),
            pltpu.SemaphoreType.DMA,
            pltpu.SemaphoreType.DMA,
        ],
    )(scores_blk, a_flat)

    return pl.pallas_call(
        _tail_body,
        out_shape=jax.ShapeDtypeStruct((MAP_H, MAP_W), jnp.float32),
    )(p_rows, q_rows, spike_threshold)
